# Initial kernel scaffold; baseline (speedup 1.0000x reference)
import functools

import jax
import jax.numpy as jnp
from jax import lax
from jax.experimental import pallas as pl
from jax.experimental.pallas import tpu as pltpu
from jax.experimental.pallas import tpu_sc as plsc

NC = 2
NS = 16
LN = 16


def _mesh():
    return plsc.VectorSubcoreMesh(core_axis_name="c", subcore_axis_name="s")


def _make_att(E, N_pad):
    EA = E // NS
    KA = 80

    def body(attr_hbm, dst_hbm, att_hbm, e_v, dst_v, att_v, s_v, zb_v,
             idx_blk, s_shared):
        cid = lax.axis_index("c")
        sid = lax.axis_index("s")
        base = sid * EA
        pltpu.sync_copy(attr_hbm.at[pl.ds(base, EA)], e_v)
        pltpu.sync_copy(dst_hbm.at[pl.ds(base, EA)], dst_v)

        def exp_body(i, c):
            e_v[pl.ds(i * LN, LN)] = jnp.exp(e_v[pl.ds(i * LN, LN)])
            return c
        lax.fori_loop(0, EA // LN, exp_body, 0)

        rpt = N_pad // NS

        def z_body(i, c):
            zb_v[pl.ds(i * LN, LN)] = jnp.zeros((LN,), jnp.float32)
            return c
        lax.fori_loop(0, rpt // LN, z_body, 0)
        pltpu.sync_copy(zb_v, s_shared.at[pl.ds(sid * rpt, rpt)])
        plsc.subcore_barrier()

        def blk_body(b, c):
            for i in range(KA // LN):
                idx_blk[pl.ds(i * LN, LN)] = dst_v[pl.ds(b * KA + i * LN, LN)]
            pltpu.sync_copy(e_v.at[pl.ds(b * KA, KA)],
                            s_shared.at[idx_blk], add=True)
            return c
        lax.fori_loop(0, EA // KA, blk_body, 0)
        plsc.subcore_barrier()

        pltpu.sync_copy(s_shared, s_v)

        def att_body(i, c):
            idx = dst_v[pl.ds(i * LN, LN)]
            sv = plsc.load_gather(s_v, [idx])
            att_v[pl.ds(i * LN, LN)] = e_v[pl.ds(i * LN, LN)] / sv
            return c
        lax.fori_loop(0, EA // LN, att_body, 0)

        @pl.when(cid == 0)
        def _():
            pltpu.sync_copy(att_v, att_hbm.at[pl.ds(base, EA)])

    return pl.kernel(
        body,
        out_type=jax.ShapeDtypeStruct((E,), jnp.float32),
        mesh=_mesh(),
        scratch_types=[
            pltpu.VMEM((EA,), jnp.float32),
            pltpu.VMEM((EA,), jnp.int32),
            pltpu.VMEM((EA,), jnp.float32),
            pltpu.VMEM((N_pad,), jnp.float32),
            pltpu.VMEM((N_pad // NS,), jnp.float32),
            pltpu.VMEM((KA,), jnp.int32),
            pltpu.VMEM_SHARED((N_pad,), jnp.float32),
        ],
    )


def _make_aggr(E, N, D, N_pad):
    ECW = E // (NC * NS)
    KC = 80

    def body(h_hbm, src_hbm, dst_hbm, att_hbm, out_hbm,
             src_v, dst_v, att_v, dst_blk, rows_v, accum):
        cid = lax.axis_index("c")
        sid = lax.axis_index("s")
        wid = sid * NC + cid
        base = wid * ECW
        rpt = N_pad // NS

        pltpu.sync_copy(src_hbm.at[pl.ds(base, ECW)], src_v)
        pltpu.sync_copy(dst_hbm.at[pl.ds(base, ECW)], dst_v)
        pltpu.sync_copy(att_hbm.at[pl.ds(base, ECW)], att_v)

        def zr_body(r, c):
            for j in range(D // LN):
                rows_v[r, pl.ds(j * LN, LN)] = jnp.zeros((LN,), jnp.float32)
            return c
        lax.fori_loop(0, KC, zr_body, 0)
        for k in range(rpt // KC):
            pltpu.sync_copy(rows_v,
                            accum.at[pl.ds(sid * rpt + k * KC, KC), :])
        plsc.subcore_barrier()

        def blk_body(b, c):
            pltpu.sync_copy(h_hbm.at[src_v.at[pl.ds(b * KC, KC)]], rows_v)
            for i in range(KC // LN):
                dst_blk[pl.ds(i * LN, LN)] = dst_v[pl.ds(b * KC + i * LN, LN)]

            def scale(e, c2):
                a = plsc.load_gather(
                    att_v, [jnp.zeros((LN,), jnp.int32) + (b * KC + e)])
                for j in range(D // LN):
                    rows_v[e, pl.ds(j * LN, LN)] = (
                        rows_v[e, pl.ds(j * LN, LN)] * a)
                return c2
            lax.fori_loop(0, KC, scale, 0)
            pltpu.sync_copy(rows_v, accum.at[dst_blk], add=True)
            return c
        lax.fori_loop(0, ECW // KC, blk_body, 0)
        plsc.subcore_barrier()

        pltpu.sync_copy(accum.at[pl.ds(sid * rpt, rpt), :],
                        out_hbm.at[cid, pl.ds(sid * rpt, rpt), :])

    return pl.kernel(
        body,
        out_type=jax.ShapeDtypeStruct((NC, N_pad, D), jnp.float32),
        mesh=_mesh(),
        scratch_types=[
            pltpu.VMEM((ECW,), jnp.int32),
            pltpu.VMEM((ECW,), jnp.int32),
            pltpu.VMEM((ECW,), jnp.float32),
            pltpu.VMEM((KC,), jnp.int32),
            pltpu.VMEM((KC, D), jnp.float32),
            pltpu.VMEM_SHARED((N_pad, D), jnp.float32),
        ],
    )


_BR = 2000


def _mm_body(x_ref, w_ref, o_ref):
    o_ref[...] = lax.dot_general(
        x_ref[...], w_ref[...], (((1,), (1,)), ((), ())),
        preferred_element_type=jnp.float32)


def _matmul(x, W):
    n, d = x.shape
    return pl.pallas_call(
        _mm_body,
        grid=(n // _BR,),
        in_specs=[pl.BlockSpec((_BR, d), lambda i: (i, 0)),
                  pl.BlockSpec((d, d), lambda i: (0, 0))],
        out_specs=pl.BlockSpec((_BR, d), lambda i: (i, 0)),
        out_shape=jax.ShapeDtypeStruct((n, d), jnp.float32),
    )(x, W)


def _upd_mm_body(p_ref, h_ref, b_ref, w_ref, o_ref):
    s = jax.nn.sigmoid(p_ref[0] + p_ref[1] + h_ref[...] + b_ref[...])
    o_ref[...] = lax.dot_general(
        s, w_ref[...], (((1,), (1,)), ((), ())),
        preferred_element_type=jnp.float32)


def _upd_matmul(p, h, b, W):
    n, d = h.shape
    return pl.pallas_call(
        _upd_mm_body,
        grid=(n // _BR,),
        in_specs=[pl.BlockSpec((NC, _BR, d), lambda i: (0, i, 0)),
                  pl.BlockSpec((_BR, d), lambda i: (i, 0)),
                  pl.BlockSpec((1, d), lambda i: (0, 0)),
                  pl.BlockSpec((d, d), lambda i: (0, 0))],
        out_specs=pl.BlockSpec((_BR, d), lambda i: (i, 0)),
        out_shape=jax.ShapeDtypeStruct((n, d), jnp.float32),
    )(p, h, b, W)


def _upd_sig_body(p_ref, h_ref, b_ref, o_ref):
    o_ref[...] = jax.nn.sigmoid(p_ref[0] + p_ref[1] + h_ref[...] + b_ref[...])


def _upd_sig(p, h, b):
    n, d = h.shape
    return pl.pallas_call(
        _upd_sig_body,
        grid=(n // _BR,),
        in_specs=[pl.BlockSpec((NC, _BR, d), lambda i: (0, i, 0)),
                  pl.BlockSpec((_BR, d), lambda i: (i, 0)),
                  pl.BlockSpec((1, d), lambda i: (0, 0))],
        out_specs=pl.BlockSpec((_BR, d), lambda i: (i, 0)),
        out_shape=jax.ShapeDtypeStruct((n, d), jnp.float32),
    )(p, h, b)


def kernel(x, edge_index, edge_attr, W1, b1, W2, b2):
    N, D = x.shape
    E = edge_attr.shape[0]
    N_pad = ((N + NS * LN * 8 - 1) // (NS * LN * 8)) * (NS * LN * 8)

    src = edge_index[0]
    dst = edge_index[1]
    b1r = b1.reshape(1, D)
    b2r = b2.reshape(1, D)

    att = _make_att(E, N_pad)(edge_attr, dst)
    h1 = _matmul(x, W1)
    aggr = _make_aggr(E, N, D, N_pad)
    p1 = aggr(h1, src, dst, att)
    h2 = _upd_matmul(p1, h1, b1r, W2)
    p2 = aggr(h2, src, dst, att)
    return _upd_sig(p2, h2, b2r)


# speedup vs baseline: 11.1110x; 11.1110x over previous
import functools

import jax
import jax.numpy as jnp
from jax import lax
from jax.experimental import pallas as pl
from jax.experimental.pallas import tpu as pltpu
from jax.experimental.pallas import tpu_sc as plsc

NC = 2
NS = 16
LN = 16


def _mesh():
    return plsc.VectorSubcoreMesh(core_axis_name="c", subcore_axis_name="s",
                                  num_cores=NC, num_subcores=NS)


def _make_att(E, N_pad):
    EA = E // NS
    KA = 80

    def body(attr_hbm, dst_hbm, att_hbm, e_v, dst_v, att_v, s_v, zb_v,
             idx_blk, s_shared):
        cid = lax.axis_index("c")
        sid = lax.axis_index("s")
        base = sid * EA
        pltpu.sync_copy(attr_hbm.at[pl.ds(base, EA)], e_v)
        pltpu.sync_copy(dst_hbm.at[pl.ds(base, EA)], dst_v)

        def exp_body(i, c):
            e_v[pl.ds(i * LN, LN)] = jnp.exp(e_v[pl.ds(i * LN, LN)])
            return c
        lax.fori_loop(0, EA // LN, exp_body, 0)

        rpt = N_pad // NS

        def z_body(i, c):
            zb_v[pl.ds(i * LN, LN)] = jnp.zeros((LN,), jnp.float32)
            return c
        lax.fori_loop(0, rpt // LN, z_body, 0)
        pltpu.sync_copy(zb_v, s_shared.at[pl.ds(sid * rpt, rpt)])
        plsc.subcore_barrier()

        def blk_body(b, c):
            for i in range(KA // LN):
                idx_blk[pl.ds(i * LN, LN)] = dst_v[pl.ds(b * KA + i * LN, LN)]
            pltpu.sync_copy(e_v.at[pl.ds(b * KA, KA)],
                            s_shared.at[idx_blk], add=True)
            return c
        lax.fori_loop(0, EA // KA, blk_body, 0)
        plsc.subcore_barrier()

        pltpu.sync_copy(s_shared, s_v)

        def att_body(i, c):
            idx = dst_v[pl.ds(i * LN, LN)]
            sv = plsc.load_gather(s_v, [idx])
            att_v[pl.ds(i * LN, LN)] = e_v[pl.ds(i * LN, LN)] / sv
            return c
        lax.fori_loop(0, EA // LN, att_body, 0)

        @pl.when(cid == 0)
        def _():
            pltpu.sync_copy(att_v, att_hbm.at[pl.ds(base, EA)])

    return pl.kernel(
        body,
        out_type=jax.ShapeDtypeStruct((E,), jnp.float32),
        mesh=_mesh(),
        compiler_params=pltpu.CompilerParams(needs_layout_passes=False),
        scratch_types=[
            pltpu.VMEM((EA,), jnp.float32),
            pltpu.VMEM((EA,), jnp.int32),
            pltpu.VMEM((EA,), jnp.float32),
            pltpu.VMEM((N_pad,), jnp.float32),
            pltpu.VMEM((N_pad // NS,), jnp.float32),
            pltpu.VMEM((KA,), jnp.int32),
            pltpu.VMEM_SHARED((N_pad,), jnp.float32),
        ],
    )


def _make_aggr(E, N, D, N_pad):
    ECW = E // (NC * NS)
    KC = 80

    def body(h_hbm, src_hbm, dst_hbm, att_hbm, out_hbm,
             src_v, dst_v, att_v, dst_blk, rows_v, accum):
        cid = lax.axis_index("c")
        sid = lax.axis_index("s")
        wid = sid * NC + cid
        base = wid * ECW
        rpt = N_pad // NS

        pltpu.sync_copy(src_hbm.at[pl.ds(base, ECW)], src_v)
        pltpu.sync_copy(dst_hbm.at[pl.ds(base, ECW)], dst_v)
        pltpu.sync_copy(att_hbm.at[pl.ds(base, ECW)], att_v)

        def zr_body(r, c):
            for j in range(D // LN):
                rows_v[r, pl.ds(j * LN, LN)] = jnp.zeros((LN,), jnp.float32)
            return c
        lax.fori_loop(0, KC, zr_body, 0)
        for k in range(rpt // KC):
            pltpu.sync_copy(rows_v,
                            accum.at[pl.ds(sid * rpt + k * KC, KC), :])
        plsc.subcore_barrier()

        def blk_body(b, c):
            pltpu.sync_copy(h_hbm.at[src_v.at[pl.ds(b * KC, KC)]], rows_v)
            for i in range(KC // LN):
                dst_blk[pl.ds(i * LN, LN)] = dst_v[pl.ds(b * KC + i * LN, LN)]

            def scale(e, c2):
                a = plsc.load_gather(
                    att_v, [jnp.zeros((LN,), jnp.int32) + (b * KC + e)])
                for j in range(D // LN):
                    rows_v[e, pl.ds(j * LN, LN)] = (
                        rows_v[e, pl.ds(j * LN, LN)] * a)
                return c2
            lax.fori_loop(0, KC, scale, 0)
            pltpu.sync_copy(rows_v, accum.at[dst_blk], add=True)
            return c
        lax.fori_loop(0, ECW // KC, blk_body, 0)
        plsc.subcore_barrier()

        pltpu.sync_copy(accum.at[pl.ds(sid * rpt, rpt), :],
                        out_hbm.at[cid, pl.ds(sid * rpt, rpt), :])

    return pl.kernel(
        body,
        out_type=jax.ShapeDtypeStruct((NC, N_pad, D), jnp.float32),
        mesh=_mesh(),
        compiler_params=pltpu.CompilerParams(needs_layout_passes=False),
        scratch_types=[
            pltpu.VMEM((ECW,), jnp.int32),
            pltpu.VMEM((ECW,), jnp.int32),
            pltpu.VMEM((ECW,), jnp.float32),
            pltpu.VMEM((KC,), jnp.int32),
            pltpu.VMEM((KC, D), jnp.float32),
            pltpu.VMEM_SHARED((N_pad, D), jnp.float32),
        ],
    )


_BR = 2000


def _mm_body(x_ref, w_ref, o_ref):
    o_ref[...] = lax.dot_general(
        x_ref[...], w_ref[...], (((1,), (1,)), ((), ())),
        preferred_element_type=jnp.float32)


def _matmul(x, W):
    n, d = x.shape
    return pl.pallas_call(
        _mm_body,
        grid=(n // _BR,),
        in_specs=[pl.BlockSpec((_BR, d), lambda i: (i, 0)),
                  pl.BlockSpec((d, d), lambda i: (0, 0))],
        out_specs=pl.BlockSpec((_BR, d), lambda i: (i, 0)),
        out_shape=jax.ShapeDtypeStruct((n, d), jnp.float32),
    )(x, W)


def _upd_mm_body(p_ref, h_ref, b_ref, w_ref, o_ref):
    s = jax.nn.sigmoid(p_ref[0] + p_ref[1] + h_ref[...] + b_ref[...])
    o_ref[...] = lax.dot_general(
        s, w_ref[...], (((1,), (1,)), ((), ())),
        preferred_element_type=jnp.float32)


def _upd_matmul(p, h, b, W):
    n, d = h.shape
    return pl.pallas_call(
        _upd_mm_body,
        grid=(n // _BR,),
        in_specs=[pl.BlockSpec((NC, _BR, d), lambda i: (0, i, 0)),
                  pl.BlockSpec((_BR, d), lambda i: (i, 0)),
                  pl.BlockSpec((1, d), lambda i: (0, 0)),
                  pl.BlockSpec((d, d), lambda i: (0, 0))],
        out_specs=pl.BlockSpec((_BR, d), lambda i: (i, 0)),
        out_shape=jax.ShapeDtypeStruct((n, d), jnp.float32),
    )(p, h, b, W)


def _upd_sig_body(p_ref, h_ref, b_ref, o_ref):
    o_ref[...] = jax.nn.sigmoid(p_ref[0] + p_ref[1] + h_ref[...] + b_ref[...])


def _upd_sig(p, h, b):
    n, d = h.shape
    return pl.pallas_call(
        _upd_sig_body,
        grid=(n // _BR,),
        in_specs=[pl.BlockSpec((NC, _BR, d), lambda i: (0, i, 0)),
                  pl.BlockSpec((_BR, d), lambda i: (i, 0)),
                  pl.BlockSpec((1, d), lambda i: (0, 0))],
        out_specs=pl.BlockSpec((_BR, d), lambda i: (i, 0)),
        out_shape=jax.ShapeDtypeStruct((n, d), jnp.float32),
    )(p, h, b)


def kernel(x, edge_index, edge_attr, W1, b1, W2, b2):
    N, D = x.shape
    E = edge_attr.shape[0]
    N_pad = ((N + NS * LN * 8 - 1) // (NS * LN * 8)) * (NS * LN * 8)

    src = edge_index[0]
    dst = edge_index[1]
    b1r = b1.reshape(1, D)
    b2r = b2.reshape(1, D)

    att = _make_att(E, N_pad)(edge_attr, dst)
    h1 = _matmul(x, W1)
    aggr = _make_aggr(E, N, D, N_pad)
    p1 = aggr(h1, src, dst, att)
    h2 = _upd_matmul(p1, h1, b1r, W2)
    p2 = aggr(h2, src, dst, att)
    return _upd_sig(p2, h2, b2r)
